# Initial kernel scaffold; baseline (speedup 1.0000x reference)
import jax
import jax.numpy as jnp
from jax.experimental import pallas as pl


def kernel(e_feats, edge_dst, W1, b1, ln_g, ln_b, W2, b2, W3, b3):
    raise NotImplementedError("write your pallas kernel here")


# speedup vs baseline: 16.0775x; 16.0775x over previous
import functools

import jax
import jax.numpy as jnp
from jax import lax
from jax.experimental import pallas as pl
from jax.experimental.pallas import tpu as pltpu
from jax.experimental.pallas import tpu_sc as plsc

E = 1600000
N = 50000
K = 8

NW = 32
C = E // NW
SUB = 10000
NCHUNK = C // SUB
W = 2000
LBUF = SUB + W + 32
OBUF = C + W + 16
SLEN = SUB + W + 32
PAD_BACK = 2024

NEG_INF = -3.0e38
INVALID_KEY = 3.0e38


def _mlp_body(x_ref, w1_ref, b1_ref, g_ref, bt_ref, w2_ref, b2_ref, w3_ref,
              b3_ref, out_ref):
    x = x_ref[...]
    h = jnp.dot(x, w1_ref[...], preferred_element_type=jnp.float32) + b1_ref[...]
    mu = jnp.mean(h, axis=1, keepdims=True)
    var = jnp.mean((h - mu) ** 2, axis=1, keepdims=True)
    h = (h - mu) / jnp.sqrt(var + 1e-5) * g_ref[...] + bt_ref[...]
    h2 = jnp.dot(h, w2_ref[...], preferred_element_type=jnp.float32) + b2_ref[...]
    a = h2[:, :4]
    g = h2[:, 4:]
    glu = a * jax.nn.sigmoid(g)
    sc = jnp.dot(glu, w3_ref[...], preferred_element_type=jnp.float32) + b3_ref[...]
    out_ref[...] = sc[:, 0].reshape(1, 1, out_ref.shape[2])


def _scores_tc(e_feats, W1, b1, ln_g, ln_b, W2, b2, W3, b3):
    R = 12800
    G = E // R
    rep = lambda i: (0, 0)
    out = pl.pallas_call(
        _mlp_body,
        grid=(G,),
        in_specs=[
            pl.BlockSpec((R, 16), lambda i: (i, 0)),
            pl.BlockSpec((16, 16), rep),
            pl.BlockSpec((16,), lambda i: (0,)),
            pl.BlockSpec((16,), lambda i: (0,)),
            pl.BlockSpec((16,), lambda i: (0,)),
            pl.BlockSpec((16, 8), rep),
            pl.BlockSpec((8,), lambda i: (0,)),
            pl.BlockSpec((4, 1), rep),
            pl.BlockSpec((1,), lambda i: (0,)),
        ],
        out_specs=pl.BlockSpec((1, 1, R), lambda i: (i, 0, 0)),
        out_shape=jax.ShapeDtypeStruct((G, 1, R), jnp.float32),
    )(e_feats, W1, b1, ln_g, ln_b, W2, b2, W3, b3)
    return out.reshape(E)


def _seg_body(s_hbm, g_hbm, out_hbm, ovh_hbm, sbuf, gbuf, obuf, starts):
    wid = lax.axis_index("s") * 2 + lax.axis_index("c")
    iota = lax.iota(jnp.int32, 16)
    lane_lo8 = iota < 8
    zeros16 = jnp.zeros((16,), jnp.float32)

    def _init(i, _):
        obuf[pl.ds(16 * i, 16)] = zeros16
        return 0
    lax.fori_loop(0, OBUF // 16, _init, 0)

    def _chunk(c, _):
        base = wid * C + c * SUB
        pltpu.sync_copy(s_hbm.at[pl.ds(base, LBUF)], sbuf)
        pltpu.sync_copy(g_hbm.at[pl.ds(base, LBUF)], gbuf)

        def _bnd(v, cur):
            g_cur = gbuf[pl.ds(8 + 16 * v, 16)]
            g_prev = gbuf[pl.ds(7 + 16 * v, 16)]
            b = g_cur != g_prev
            bi = b.astype(jnp.int32)
            cs = plsc.cumsum(bi)
            idx = jnp.where(b, cur + cs - 1, SLEN - 1)
            plsc.store_scatter(starts, [idx], iota + 16 * v)
            return cur + jnp.sum(bi)

        n_owned = lax.fori_loop(0, SUB // 16, _bnd, jnp.int32(0))
        cursor = lax.fori_loop(SUB // 16, (SUB + W) // 16, _bnd, n_owned)
        starts[pl.ds(cursor, 16)] = jnp.full((16,), SUB + W, jnp.int32)

        def _segment(k, _):
            sv = starts[pl.ds(k, 16)]
            p0 = sv[0]
            p1 = sv[1]
            nv = (p1 - p0 + 15) // 16

            def _valid(i):
                return (p0 + 16 * i + iota) < p1

            def _pa(i, mv):
                x = sbuf[pl.ds(8 + p0 + 16 * i, 16)]
                return jnp.maximum(mv, jnp.where(_valid(i), x, NEG_INF))
            m_vec = lax.fori_loop(0, nv, _pa, jnp.full((16,), NEG_INF))
            m_spl = jnp.broadcast_to(jnp.max(m_vec), (16,))

            def _pb(i, carry):
                s_vec, top = carry
                x = sbuf[pl.ds(8 + p0 + 16 * i, 16)]
                ex = jnp.exp(x - m_spl)
                val = _valid(i)
                s_vec = s_vec + jnp.where(val, ex, 0.0)
                key = jnp.where(val, -x, INVALID_KEY)
                cand = lax.sort(key, dimension=0)
                merged = jnp.where(lane_lo8, top, lax.rev(cand, (0,)))
                top = lax.sort(merged, dimension=0)
                return s_vec, top
            s_vec, top = lax.fori_loop(
                0, nv, _pb,
                (zeros16, jnp.full((16,), INVALID_KEY)))
            s_spl = jnp.broadcast_to(jnp.sum(s_vec), (16,))
            tneg = jnp.broadcast_to(
                jnp.max(jnp.where(iota == 7, top, NEG_INF)), (16,))

            def _pc(i, c1):
                x = sbuf[pl.ds(8 + p0 + 16 * i, 16)]
                key = jnp.where(_valid(i), -x, INVALID_KEY)
                return c1 + jnp.sum((key < tneg).astype(jnp.int32))
            c1 = lax.fori_loop(0, nv, _pc, jnp.int32(0))
            budget = jnp.broadcast_to(jnp.int32(K) - c1, (16,))

            def _pd(i, carry_eq):
                val = _valid(i)
                x = sbuf[pl.ds(8 + p0 + 16 * i, 16)]
                ex = jnp.exp(x - m_spl)
                key = jnp.where(val, -x, INVALID_KEY)
                gt = key < tneg
                eq = (key == tneg) & val
                rank = plsc.cumsum(eq.astype(jnp.int32)) + carry_eq
                keep = gt | (eq & (rank <= budget))
                w = ex / s_spl
                outv = jnp.where(keep, w, 0.0)
                off = c * SUB + p0 + 16 * i
                old = obuf[pl.ds(off, 16)]
                obuf[pl.ds(off, 16)] = jnp.where(val, outv, old)
                return carry_eq + jnp.sum(eq.astype(jnp.int32))
            lax.fori_loop(0, nv, _pd, jnp.int32(0))
            return 0

        lax.fori_loop(0, n_owned, _segment, 0)
        return 0

    lax.fori_loop(0, NCHUNK, _chunk, 0)

    pltpu.sync_copy(obuf.at[pl.ds(0, C)], out_hbm.at[pl.ds(wid * C, C)])
    pltpu.sync_copy(obuf.at[pl.ds(C, W)], ovh_hbm.at[pl.ds(wid * W, W)])


_seg_sc = functools.partial(
    pl.kernel,
    out_type=[jax.ShapeDtypeStruct((E,), jnp.float32),
              jax.ShapeDtypeStruct((NW * W,), jnp.float32)],
    mesh=plsc.VectorSubcoreMesh(core_axis_name="c", subcore_axis_name="s",
                                num_cores=2, num_subcores=16),
    scratch_types=[pltpu.VMEM((LBUF,), jnp.float32),
                   pltpu.VMEM((LBUF,), jnp.int32),
                   pltpu.VMEM((OBUF,), jnp.float32),
                   pltpu.VMEM((SLEN,), jnp.int32)],
    compiler_params=pltpu.CompilerParams(needs_layout_passes=False),
)(_seg_body)


@jax.jit
def kernel(e_feats, edge_dst, W1, b1, ln_g, ln_b, W2, b2, W3, b3):
    scores = _scores_tc(e_feats, W1, b1, ln_g, ln_b, W2, b2, W3, b3)
    seg = edge_dst.astype(jnp.int32)
    s_pad = jnp.concatenate(
        [jnp.zeros((8,), jnp.float32), scores, jnp.zeros((PAD_BACK,), jnp.float32)])
    g_pad = jnp.concatenate(
        [jnp.full((8,), -1, jnp.int32), seg, jnp.full((PAD_BACK,), N, jnp.int32)])
    out_main, ovh = _seg_sc(s_pad, g_pad)
    shifted = jnp.pad(ovh.reshape(NW, W), ((1, 0), (0, C - W)))[:NW]
    return (out_main.reshape(NW, C) + shifted).reshape(E)


# speedup vs baseline: 17.0166x; 1.0584x over previous
import functools

import jax
import jax.numpy as jnp
from jax import lax
from jax.experimental import pallas as pl
from jax.experimental.pallas import tpu as pltpu
from jax.experimental.pallas import tpu_sc as plsc

E = 1600000
N = 50000
K = 8

NW = 32
C = E // NW
SUB = 10000
NCHUNK = C // SUB
W = 2000
LBUF = SUB + W + 32
OBUF = C + W + 32
TRASH = C + W + 16
SLEN = SUB + W + 32
PAD_BACK = 2024

NEG_INF = -3.0e38
INVALID_KEY = 3.0e38


def _mlp_body(x_ref, w1_ref, b1_ref, g_ref, bt_ref, w2_ref, b2_ref, w3_ref,
              b3_ref, out_ref):
    x = x_ref[...]
    h = jnp.dot(x, w1_ref[...], preferred_element_type=jnp.float32) + b1_ref[...]
    mu = jnp.mean(h, axis=1, keepdims=True)
    var = jnp.mean((h - mu) ** 2, axis=1, keepdims=True)
    h = (h - mu) / jnp.sqrt(var + 1e-5) * g_ref[...] + bt_ref[...]
    h2 = jnp.dot(h, w2_ref[...], preferred_element_type=jnp.float32) + b2_ref[...]
    a = h2[:, :4]
    g = h2[:, 4:]
    glu = a * jax.nn.sigmoid(g)
    sc = jnp.dot(glu, w3_ref[...], preferred_element_type=jnp.float32) + b3_ref[...]
    out_ref[...] = sc[:, 0].reshape(1, 1, out_ref.shape[2])


def _scores_tc(e_feats, W1, b1, ln_g, ln_b, W2, b2, W3, b3):
    R = 12800
    G = E // R
    rep = lambda i: (0, 0)
    out = pl.pallas_call(
        _mlp_body,
        grid=(G,),
        in_specs=[
            pl.BlockSpec((R, 16), lambda i: (i, 0)),
            pl.BlockSpec((16, 16), rep),
            pl.BlockSpec((16,), lambda i: (0,)),
            pl.BlockSpec((16,), lambda i: (0,)),
            pl.BlockSpec((16,), lambda i: (0,)),
            pl.BlockSpec((16, 8), rep),
            pl.BlockSpec((8,), lambda i: (0,)),
            pl.BlockSpec((4, 1), rep),
            pl.BlockSpec((1,), lambda i: (0,)),
        ],
        out_specs=pl.BlockSpec((1, 1, R), lambda i: (i, 0, 0)),
        out_shape=jax.ShapeDtypeStruct((G, 1, R), jnp.float32),
    )(e_feats, W1, b1, ln_g, ln_b, W2, b2, W3, b3)
    return out.reshape(E)


def _seg_body(s_hbm, g_hbm, out_hbm, ovh_hbm, sbuf, gbuf, obuf, starts):
    wid = lax.axis_index("s") * 2 + lax.axis_index("c")
    iota = lax.iota(jnp.int32, 16)
    lane_lo8 = iota < 8
    zeros16 = jnp.zeros((16,), jnp.float32)

    def _init(i, _):
        obuf[pl.ds(16 * i, 16)] = zeros16
        return 0
    lax.fori_loop(0, OBUF // 16, _init, 0)

    def _chunk(c, _):
        base = wid * C + c * SUB
        pltpu.sync_copy(s_hbm.at[pl.ds(base, LBUF)], sbuf)
        pltpu.sync_copy(g_hbm.at[pl.ds(base, LBUF)], gbuf)

        def _bnd(v, cur):
            g_cur = gbuf[pl.ds(8 + 16 * v, 16)]
            g_prev = gbuf[pl.ds(7 + 16 * v, 16)]
            b = g_cur != g_prev
            bi = b.astype(jnp.int32)
            cs = plsc.cumsum(bi)
            idx = jnp.where(b, cur + cs - 1, SLEN - 1)
            plsc.store_scatter(starts, [idx], iota + 16 * v)
            return cur + jnp.sum(bi)

        n_owned = lax.fori_loop(0, SUB // 16, _bnd, jnp.int32(0))
        cursor = lax.fori_loop(SUB // 16, (SUB + W) // 16, _bnd, n_owned)
        starts[pl.ds(cursor, 16)] = jnp.full((16,), SUB + W, jnp.int32)

        def _group(g, _):
            k0 = 16 * g
            p0v = starts[pl.ds(k0, 16)]
            p1v = starts[pl.ds(k0 + 1, 16)]
            lm = (k0 + iota) < n_owned
            lenv = jnp.where(lm, p1v - p0v, 0)
            nvm = jnp.max(lenv)
            a0 = p0v + 8

            def _p1(j, Ts):
                em = j < lenv
                addr = jnp.where(em, a0 + j, 0)
                x = plsc.load_gather(sbuf, [addr])
                v = jnp.where(em, x, NEG_INF)
                out = []
                for tr in Ts:
                    hi = jnp.maximum(tr, v)
                    v = jnp.minimum(tr, v)
                    out.append(hi)
                return tuple(out)
            Ts = lax.fori_loop(
                0, nvm, _p1,
                tuple(jnp.full((16,), NEG_INF) for _ in range(K)))
            m = Ts[0]
            thr = Ts[K - 1]
            c1 = jnp.zeros((16,), jnp.int32)
            for tr in Ts[:K - 1]:
                c1 = c1 + (tr > thr).astype(jnp.int32)
            budget = K - c1

            def _p2(j, s_vec):
                em = j < lenv
                addr = jnp.where(em, a0 + j, 0)
                x = plsc.load_gather(sbuf, [addr])
                return s_vec + jnp.where(em, jnp.exp(x - m), 0.0)
            s_vec = lax.fori_loop(0, nvm, _p2, zeros16)
            rcp = 1.0 / jnp.where(lenv > 0, s_vec, 1.0)
            ob = c * SUB + p0v

            def _p3(j, ceq):
                em = j < lenv
                addr = jnp.where(em, a0 + j, 0)
                x = plsc.load_gather(sbuf, [addr])
                wv = jnp.exp(x - m) * rcp
                gt = (x > thr) & em
                eq = (x == thr) & em
                ceq = ceq + eq.astype(jnp.int32)
                keep = gt | (eq & (ceq <= budget))
                oidx = jnp.where(keep, ob + j, TRASH)
                plsc.store_scatter(obuf, [oidx], wv)
                return ceq
            lax.fori_loop(0, nvm, _p3, jnp.zeros((16,), jnp.int32))
            return 0

        lax.fori_loop(0, (n_owned + 15) // 16, _group, 0)
        return 0

    lax.fori_loop(0, NCHUNK, _chunk, 0)

    pltpu.sync_copy(obuf.at[pl.ds(0, C)], out_hbm.at[pl.ds(wid * C, C)])
    pltpu.sync_copy(obuf.at[pl.ds(C, W)], ovh_hbm.at[pl.ds(wid * W, W)])


_seg_sc = functools.partial(
    pl.kernel,
    out_type=[jax.ShapeDtypeStruct((E,), jnp.float32),
              jax.ShapeDtypeStruct((NW * W,), jnp.float32)],
    mesh=plsc.VectorSubcoreMesh(core_axis_name="c", subcore_axis_name="s",
                                num_cores=2, num_subcores=16),
    scratch_types=[pltpu.VMEM((LBUF,), jnp.float32),
                   pltpu.VMEM((LBUF,), jnp.int32),
                   pltpu.VMEM((OBUF,), jnp.float32),
                   pltpu.VMEM((SLEN,), jnp.int32)],
    compiler_params=pltpu.CompilerParams(needs_layout_passes=False),
)(_seg_body)


@jax.jit
def kernel(e_feats, edge_dst, W1, b1, ln_g, ln_b, W2, b2, W3, b3):
    scores = _scores_tc(e_feats, W1, b1, ln_g, ln_b, W2, b2, W3, b3)
    seg = edge_dst.astype(jnp.int32)
    s_pad = jnp.concatenate(
        [jnp.zeros((8,), jnp.float32), scores, jnp.zeros((PAD_BACK,), jnp.float32)])
    g_pad = jnp.concatenate(
        [jnp.full((8,), -1, jnp.int32), seg, jnp.full((PAD_BACK,), N, jnp.int32)])
    out_main, ovh = _seg_sc(s_pad, g_pad)
    shifted = jnp.pad(ovh.reshape(NW, W), ((1, 0), (0, C - W)))[:NW]
    return (out_main.reshape(NW, C) + shifted).reshape(E)
